# Initial kernel scaffold; baseline (speedup 1.0000x reference)
import functools

import jax
import jax.numpy as jnp
from jax import lax
from jax.experimental import pallas as pl
from jax.experimental.pallas import tpu as pltpu
from jax.experimental.pallas import tpu_sc as plsc

N = 10000
D = 128
E = 320000
HID = 128

NC = 2
NS = 16
NW = NC * NS
EDGES_PER_W = E // NW
CHUNK = 80
NCHUNKS = EDGES_PER_W // CHUNK
ROWS_PER_S = N // NS
DEGW = 16


def _sc_aggregate(x, row, col, z128, z16, ones_h):
    mesh = plsc.VectorSubcoreMesh(core_axis_name="c", subcore_axis_name="s")

    @functools.partial(
        pl.kernel,
        out_type=(
            jax.ShapeDtypeStruct((NC, N, D), jnp.float32),
            jax.ShapeDtypeStruct((NC, N, DEGW), jnp.float32),
        ),
        mesh=mesh,
        scratch_types=[
            pltpu.VMEM_SHARED((N, D), jnp.float32),
            pltpu.VMEM_SHARED((N, DEGW), jnp.float32),
            pltpu.VMEM((CHUNK,), jnp.int32),
            pltpu.VMEM((CHUNK,), jnp.int32),
            pltpu.VMEM((CHUNK, D), jnp.float32),
            pltpu.VMEM((CHUNK, DEGW), jnp.float32),
            pltpu.SemaphoreType.DMA,
        ],
    )
    def k(x_hbm, row_hbm, col_hbm, z128_hbm, z16_hbm, ones_hbm,
          agg_out, deg_out,
          agg_sh, deg_sh, row_v, col_v, rows_v, ones_v, sem):
        c = lax.axis_index("c")
        s = lax.axis_index("s")
        wid = s * NC + c

        r0 = s * ROWS_PER_S
        pltpu.sync_copy(z128_hbm.at[pl.ds(r0, ROWS_PER_S)],
                        agg_sh.at[pl.ds(r0, ROWS_PER_S)])
        pltpu.sync_copy(z16_hbm.at[pl.ds(r0, ROWS_PER_S)],
                        deg_sh.at[pl.ds(r0, ROWS_PER_S)])
        pltpu.sync_copy(ones_hbm, ones_v)
        plsc.subcore_barrier()

        base = wid * EDGES_PER_W

        def body(j, carry):
            off = base + j * CHUNK
            pltpu.sync_copy(row_hbm.at[pl.ds(off, CHUNK)], row_v)
            pltpu.sync_copy(col_hbm.at[pl.ds(off, CHUNK)], col_v)
            pltpu.async_copy(x_hbm.at[col_v], rows_v, sem).wait()
            pltpu.sync_copy(rows_v, agg_sh.at[row_v], add=True)
            pltpu.sync_copy(ones_v, deg_sh.at[row_v], add=True)
            return carry

        lax.fori_loop(0, NCHUNKS, body, 0)
        plsc.subcore_barrier()

        pltpu.sync_copy(agg_sh.at[pl.ds(r0, ROWS_PER_S)],
                        agg_out.at[c, pl.ds(r0, ROWS_PER_S)])
        pltpu.sync_copy(deg_sh.at[pl.ds(r0, ROWS_PER_S)],
                        deg_out.at[c, pl.ds(r0, ROWS_PER_S)])

    return k(x, row, col, z128, z16, ones_h)


TC_ROWS = 1000


def _tc_mlp_body(x_ref, agg_ref, deg_ref, w1a_ref, w1b_ref, w2_ref, w3_ref,
                 w4_ref, b1_ref, b2_ref, b3_ref, b4_ref, out_ref):
    deg = deg_ref[0][:, 0:1] + deg_ref[1][:, 0:1]
    agg = (agg_ref[0] + agg_ref[1]) * (1.0 / jnp.maximum(deg, 1.0))
    f32 = jnp.float32
    h = jnp.maximum(
        jnp.dot(x_ref[...], w1a_ref[...], preferred_element_type=f32)
        + jnp.dot(agg, w1b_ref[...], preferred_element_type=f32)
        + b1_ref[...], 0.0)
    h = jnp.maximum(
        jnp.dot(h, w2_ref[...], preferred_element_type=f32) + b2_ref[...], 0.0)
    h = jnp.maximum(
        jnp.dot(h, w3_ref[...], preferred_element_type=f32) + b3_ref[...], 0.0)
    out_ref[...] = (
        jnp.dot(h, w4_ref[...], preferred_element_type=f32) + b4_ref[...])


def _tc_mlp(x, agg_p, deg_p, w1a, w1b, w2, w3, w4, b1, b2, b3, b4):
    grid = (N // TC_ROWS,)
    full = lambda shape: pl.BlockSpec(shape, lambda i: (0,) * len(shape))
    return pl.pallas_call(
        _tc_mlp_body,
        grid=grid,
        in_specs=[
            pl.BlockSpec((TC_ROWS, D), lambda i: (i, 0)),
            pl.BlockSpec((NC, TC_ROWS, D), lambda i: (0, i, 0)),
            pl.BlockSpec((NC, TC_ROWS, DEGW), lambda i: (0, i, 0)),
            full((D, HID)), full((D, HID)), full((HID, HID)),
            full((HID, HID)), full((HID, D)),
            full((1, HID)), full((1, HID)), full((1, HID)), full((1, D)),
        ],
        out_specs=pl.BlockSpec((TC_ROWS, D), lambda i: (i, 0)),
        out_shape=jax.ShapeDtypeStruct((N, D), jnp.float32),
    )(x, agg_p, deg_p, w1a, w1b, w2, w3, w4, b1, b2, b3, b4)


def kernel(x, edge_index, W1, b1, W2, b2, W3, b3, W4, b4):
    row = edge_index[0].astype(jnp.int32)
    col = edge_index[1].astype(jnp.int32)
    z128 = jnp.zeros((N, D), jnp.float32)
    z16 = jnp.zeros((N, DEGW), jnp.float32)
    ones_h = jnp.ones((CHUNK, DEGW), jnp.float32)
    agg_p, deg_p = _sc_aggregate(x, row, col, z128, z16, ones_h)
    w1a = W1[:, :D].T
    w1b = W1[:, D:].T
    return _tc_mlp(x, agg_p, deg_p, w1a, w1b, W2.T, W3.T, W4.T,
                   b1.reshape(1, HID), b2.reshape(1, HID),
                   b3.reshape(1, HID), b4.reshape(1, D))


# speedup vs baseline: 4.8459x; 4.8459x over previous
import functools

import jax
import jax.numpy as jnp
from jax import lax
from jax.experimental import pallas as pl
from jax.experimental.pallas import tpu as pltpu
from jax.experimental.pallas import tpu_sc as plsc

N = 10000
D = 128
E = 320000
HID = 128

NC = 2
NS = 16
NW = NC * NS
LANES = 16
EDGES_PER_W = E // NW
CHUNK = 80
NCHUNKS = EDGES_PER_W // CHUNK
NPAD = 10240
ROWS_PER_S = NPAD // NS
ZCHUNKS = ROWS_PER_S // CHUNK


def _sc_aggregate(x, row, col, z128):
    mesh = plsc.VectorSubcoreMesh(core_axis_name="c", subcore_axis_name="s")

    @functools.partial(
        pl.kernel,
        out_type=(
            jax.ShapeDtypeStruct((NC, NPAD, D), jnp.float32),
            jax.ShapeDtypeStruct((NC, NS, NPAD), jnp.float32),
        ),
        mesh=mesh,
        compiler_params=pltpu.CompilerParams(needs_layout_passes=False),
        scratch_types=[
            pltpu.VMEM_SHARED((NPAD, D), jnp.float32),
            pltpu.VMEM((CHUNK,), jnp.int32),
            pltpu.VMEM((CHUNK,), jnp.int32),
            pltpu.VMEM((CHUNK, D), jnp.float32),
            pltpu.VMEM((NPAD,), jnp.float32),
            pltpu.SemaphoreType.DMA,
        ],
    )
    def k(x_hbm, row_hbm, col_hbm, z128_hbm,
          agg_out, deg_out,
          agg_sh, row_v, col_v, rows_v, deg_v, sem):
        c = lax.axis_index("c")
        s = lax.axis_index("s")
        wid = s * NC + c

        pltpu.sync_copy(z128_hbm, rows_v)
        r0 = s * ROWS_PER_S

        def zinit(i, carry):
            pltpu.sync_copy(rows_v, agg_sh.at[pl.ds(r0 + i * CHUNK, CHUNK)])
            return carry

        lax.fori_loop(0, ZCHUNKS, zinit, 0)

        zeros16 = jnp.zeros((LANES,), jnp.float32)

        def zdeg(i, carry):
            deg_v[pl.ds(i * LANES, LANES)] = zeros16
            return carry

        lax.fori_loop(0, NPAD // LANES, zdeg, 0)
        plsc.subcore_barrier()

        base = wid * EDGES_PER_W
        ones16 = jnp.full((LANES,), 1.0, jnp.float32)

        def body(j, carry):
            off = base + j * CHUNK
            pltpu.sync_copy(row_hbm.at[pl.ds(off, CHUNK)], row_v)
            pltpu.sync_copy(col_hbm.at[pl.ds(off, CHUNK)], col_v)
            pltpu.async_copy(x_hbm.at[col_v], rows_v, sem).wait()
            pltpu.sync_copy(rows_v, agg_sh.at[row_v], add=True)
            for kk in range(CHUNK // LANES):
                idx = row_v[pl.ds(kk * LANES, LANES)]
                plsc.addupdate_scatter(deg_v, [idx], ones16)
            return carry

        lax.fori_loop(0, NCHUNKS, body, 0)
        plsc.subcore_barrier()

        def wout(i, carry):
            rr = r0 + i * CHUNK
            pltpu.sync_copy(agg_sh.at[pl.ds(rr, CHUNK)], rows_v)
            pltpu.sync_copy(rows_v, agg_out.at[c, pl.ds(rr, CHUNK)])
            return carry

        lax.fori_loop(0, ZCHUNKS, wout, 0)
        pltpu.sync_copy(deg_v, deg_out.at[c, s])

    return k(x, row, col, z128)


TC_ROWS = 1000


def _tc_mlp_body(x_ref, agg_ref, deg_ref, w1a_ref, w1b_ref, w2_ref, w3_ref,
                 w4_ref, b1_ref, b2_ref, b3_ref, b4_ref, out_ref):
    deg = jnp.sum(deg_ref[...], axis=1, keepdims=True)
    agg = (agg_ref[0] + agg_ref[1]) * (1.0 / jnp.maximum(deg, 1.0))
    f32 = jnp.float32
    h = jnp.maximum(
        jnp.dot(x_ref[...], w1a_ref[...], preferred_element_type=f32)
        + jnp.dot(agg, w1b_ref[...], preferred_element_type=f32)
        + b1_ref[...], 0.0)
    h = jnp.maximum(
        jnp.dot(h, w2_ref[...], preferred_element_type=f32) + b2_ref[...], 0.0)
    h = jnp.maximum(
        jnp.dot(h, w3_ref[...], preferred_element_type=f32) + b3_ref[...], 0.0)
    out_ref[...] = (
        jnp.dot(h, w4_ref[...], preferred_element_type=f32) + b4_ref[...])


def _tc_mlp(x, agg_p, deg_t, w1a, w1b, w2, w3, w4, b1, b2, b3, b4):
    grid = (N // TC_ROWS,)
    full = lambda shape: pl.BlockSpec(shape, lambda i: (0,) * len(shape))
    return pl.pallas_call(
        _tc_mlp_body,
        grid=grid,
        in_specs=[
            pl.BlockSpec((TC_ROWS, D), lambda i: (i, 0)),
            pl.BlockSpec((NC, TC_ROWS, D), lambda i: (0, i, 0)),
            pl.BlockSpec((TC_ROWS, NW), lambda i: (i, 0)),
            full((D, HID)), full((D, HID)), full((HID, HID)),
            full((HID, HID)), full((HID, D)),
            full((1, HID)), full((1, HID)), full((1, HID)), full((1, D)),
        ],
        out_specs=pl.BlockSpec((TC_ROWS, D), lambda i: (i, 0)),
        out_shape=jax.ShapeDtypeStruct((N, D), jnp.float32),
    )(x, agg_p, deg_t, w1a, w1b, w2, w3, w4, b1, b2, b3, b4)


def kernel(x, edge_index, W1, b1, W2, b2, W3, b3, W4, b4):
    row = edge_index[0].astype(jnp.int32)
    col = edge_index[1].astype(jnp.int32)
    z128 = jnp.zeros((CHUNK, D), jnp.float32)
    agg_p, deg_p = _sc_aggregate(x, row, col, z128)
    deg_t = jnp.transpose(deg_p.reshape(NC * NS, NPAD))
    w1a = W1[:, :D].T
    w1b = W1[:, D:].T
    return _tc_mlp(x, agg_p, deg_t, w1a, w1b, W2.T, W3.T, W4.T,
                   b1.reshape(1, HID), b2.reshape(1, HID),
                   b3.reshape(1, HID), b4.reshape(1, D))


# speedup vs baseline: 6.4206x; 1.3250x over previous
import functools

import jax
import jax.numpy as jnp
from jax import lax
from jax.experimental import pallas as pl
from jax.experimental.pallas import tpu as pltpu
from jax.experimental.pallas import tpu_sc as plsc

N = 10000
D = 128
E = 320000
HID = 128

NC = 2
NS = 16
NW = NC * NS
LANES = 16
CHUNK = 80
NCH = E // CHUNK
CH_PER_W = NCH // NW
NBUF = 2
NPAD = 10240
ROWS_PER_S = NPAD // NS
ZCHUNKS = ROWS_PER_S // CHUNK


def _sc_aggregate(x, eidx, z128):
    mesh = plsc.VectorSubcoreMesh(core_axis_name="c", subcore_axis_name="s")

    @functools.partial(
        pl.kernel,
        out_type=(
            jax.ShapeDtypeStruct((NC, NPAD, D), jnp.float32),
            jax.ShapeDtypeStruct((NC, NS, NPAD), jnp.float32),
        ),
        mesh=mesh,
        compiler_params=pltpu.CompilerParams(needs_layout_passes=False),
        scratch_types=[
            pltpu.VMEM_SHARED((NPAD, D), jnp.float32),
            pltpu.VMEM((NBUF, 2, CHUNK), jnp.int32),
            pltpu.VMEM((NBUF, CHUNK, D), jnp.float32),
            pltpu.VMEM((NPAD,), jnp.float32),
            pltpu.SemaphoreType.DMA,
            pltpu.SemaphoreType.DMA,
            pltpu.SemaphoreType.DMA,
        ],
    )
    def k(x_hbm, eidx_hbm, z128_hbm,
          agg_out, deg_out,
          agg_sh, idx_v, rows_v, deg_v, gsem, ssem0, ssem1):
        c = lax.axis_index("c")
        s = lax.axis_index("s")
        wid = s * NC + c

        pltpu.sync_copy(z128_hbm, rows_v.at[0])
        r0 = s * ROWS_PER_S

        def zinit(i, carry):
            pltpu.sync_copy(rows_v.at[0],
                            agg_sh.at[pl.ds(r0 + i * CHUNK, CHUNK)])
            return carry

        lax.fori_loop(0, ZCHUNKS, zinit, 0)

        zeros16 = jnp.zeros((LANES,), jnp.float32)

        def zdeg(i, carry):
            deg_v[pl.ds(i * LANES, LANES)] = zeros16
            return carry

        lax.fori_loop(0, NPAD // LANES, zdeg, 0)
        plsc.subcore_barrier()

        cbase = wid * CH_PER_W
        ones16 = jnp.full((LANES,), 1.0, jnp.float32)

        ssems = (ssem0, ssem1)

        def chunk_step(j, b, drain):
            if drain:
                pltpu.make_async_copy(rows_v.at[b],
                                      agg_sh.at[idx_v.at[b, 0]],
                                      ssems[b]).wait()
            pltpu.sync_copy(eidx_hbm.at[cbase + j], idx_v.at[b])
            pltpu.async_copy(x_hbm.at[idx_v.at[b, 1]], rows_v.at[b],
                             gsem).wait()
            pltpu.async_copy(rows_v.at[b], agg_sh.at[idx_v.at[b, 0]],
                             ssems[b], add=True)
            for kk in range(CHUNK // LANES):
                idx = idx_v[b, 0, pl.ds(kk * LANES, LANES)]
                plsc.addupdate_scatter(deg_v, [idx], ones16)

        chunk_step(0, 0, drain=False)
        chunk_step(1, 1, drain=False)

        def body(t, carry):
            j = NBUF + t * NBUF
            chunk_step(j, 0, drain=True)
            chunk_step(j + 1, 1, drain=True)
            return carry

        npairs = (CH_PER_W - NBUF) // NBUF
        lax.fori_loop(0, npairs, body, 0)
        chunk_step(CH_PER_W - 1, 0, drain=True)
        pltpu.make_async_copy(rows_v.at[0], agg_sh.at[idx_v.at[0, 0]],
                              ssem0).wait()
        pltpu.make_async_copy(rows_v.at[1], agg_sh.at[idx_v.at[1, 0]],
                              ssem1).wait()
        plsc.subcore_barrier()

        def wout(i, carry):
            rr = r0 + i * CHUNK
            pltpu.sync_copy(agg_sh.at[pl.ds(rr, CHUNK)], rows_v.at[0])
            pltpu.sync_copy(rows_v.at[0], agg_out.at[c, pl.ds(rr, CHUNK)])
            return carry

        lax.fori_loop(0, ZCHUNKS, wout, 0)
        pltpu.sync_copy(deg_v, deg_out.at[c, s])

    return k(x, eidx, z128)


TC_ROWS = 1000


def _tc_mlp_body(x_ref, agg_ref, deg_ref, w1a_ref, w1b_ref, w2_ref, w3_ref,
                 w4_ref, b1_ref, b2_ref, b3_ref, b4_ref, out_ref):
    deg = jnp.sum(deg_ref[...], axis=1, keepdims=True)
    agg = (agg_ref[0] + agg_ref[1]) * (1.0 / jnp.maximum(deg, 1.0))
    f32 = jnp.float32
    h = jnp.maximum(
        jnp.dot(x_ref[...], w1a_ref[...], preferred_element_type=f32)
        + jnp.dot(agg, w1b_ref[...], preferred_element_type=f32)
        + b1_ref[...], 0.0)
    h = jnp.maximum(
        jnp.dot(h, w2_ref[...], preferred_element_type=f32) + b2_ref[...], 0.0)
    h = jnp.maximum(
        jnp.dot(h, w3_ref[...], preferred_element_type=f32) + b3_ref[...], 0.0)
    out_ref[...] = (
        jnp.dot(h, w4_ref[...], preferred_element_type=f32) + b4_ref[...])


def _tc_mlp(x, agg_p, deg_t, w1a, w1b, w2, w3, w4, b1, b2, b3, b4):
    grid = (N // TC_ROWS,)
    full = lambda shape: pl.BlockSpec(shape, lambda i: (0,) * len(shape))
    return pl.pallas_call(
        _tc_mlp_body,
        grid=grid,
        in_specs=[
            pl.BlockSpec((TC_ROWS, D), lambda i: (i, 0)),
            pl.BlockSpec((NC, TC_ROWS, D), lambda i: (0, i, 0)),
            pl.BlockSpec((TC_ROWS, NW), lambda i: (i, 0)),
            full((D, HID)), full((D, HID)), full((HID, HID)),
            full((HID, HID)), full((HID, D)),
            full((1, HID)), full((1, HID)), full((1, HID)), full((1, D)),
        ],
        out_specs=pl.BlockSpec((TC_ROWS, D), lambda i: (i, 0)),
        out_shape=jax.ShapeDtypeStruct((N, D), jnp.float32),
    )(x, agg_p, deg_t, w1a, w1b, w2, w3, w4, b1, b2, b3, b4)


def kernel(x, edge_index, W1, b1, W2, b2, W3, b3, W4, b4):
    eidx = jnp.transpose(
        edge_index.astype(jnp.int32).reshape(2, NCH, CHUNK), (1, 0, 2))
    z128 = jnp.zeros((CHUNK, D), jnp.float32)
    agg_p, deg_p = _sc_aggregate(x, eidx, z128)
    deg_t = jnp.transpose(deg_p.reshape(NC * NS, NPAD))
    w1a = W1[:, :D].T
    w1b = W1[:, D:].T
    return _tc_mlp(x, agg_p, deg_t, w1a, w1b, W2.T, W3.T, W4.T,
                   b1.reshape(1, HID), b2.reshape(1, HID),
                   b3.reshape(1, HID), b4.reshape(1, D))
